# baseline (speedup 1.0000x reference)
import functools

import jax
import jax.numpy as jnp
import numpy as np
from jax import lax
from jax.experimental import pallas as pl
from jax.experimental.pallas import tpu as pltpu
from jax.experimental.pallas import tpu_sc as plsc

_TAU = 1.5
_B, _C, _H, _W = 8, 10, 96, 96
_NCLS = _C - 1
_NW = 32
_RB = _H // 4
_LANES = 16
_KX = _W // _LANES
_RG = 3
_D2MAX = 2 * (_H - 1) * (_H - 1)
_TABN = ((_D2MAX + 2 + 7) // 8) * 8
_FAR = 1024
_ACC0 = 1 << 22
_CGROUPS = ((1, 2, 3, 4, 5), (6, 7, 8, 9))


def _splat_i32(x):
    return jnp.full((_LANES,), x, dtype=jnp.int32)


def _sc_body(logits_hbm, targets_hbm, dx2_hbm, tab_hbm, out_hbm,
             tgt_v, log_v, dx2_v, tab_v, d1sq_v, maxl_v, denom_v, num_v,
             out_v, sem):
    wid = lax.axis_index("s") * 2 + lax.axis_index("c")
    b = wid // 4
    row0 = (wid % 4) * _RB

    cp1 = pltpu.async_copy(targets_hbm.at[b], tgt_v, sem)
    cp2 = pltpu.async_copy(logits_hbm.at[b, :, pl.ds(row0, _RB), :], log_v,
                           sem)
    cp3 = pltpu.async_copy(dx2_hbm, dx2_v, sem)
    cp4 = pltpu.async_copy(tab_hbm, tab_v, sem)
    cp1.wait()
    cp2.wait()
    cp3.wait()
    cp4.wait()

    for grp in _CGROUPS:
        ng = len(grp)

        def fwd(y, dist):
            lbls = [tgt_v[y, pl.ds(k * _LANES, _LANES)] for k in range(_KX)]
            new = []
            for ci, c in enumerate(grp):
                for k in range(_KX):
                    new.append(jnp.where(lbls[k] == c, 0,
                                         dist[ci * _KX + k] + 1))
            rel = y - row0

            @pl.when(jnp.logical_and(rel >= 0, rel < _RB))
            def _():
                for ci, c in enumerate(grp):
                    for k in range(_KX):
                        off = ((c - 1) * _RB + rel) * _W + k * _LANES
                        d1sq_v[pl.ds(off, _LANES)] = new[ci * _KX + k]

            return tuple(new)

        lax.fori_loop(0, _H, fwd,
                      tuple(_splat_i32(_FAR) for _ in range(ng * _KX)))

        def bwd(i, dist):
            y = (_H - 1) - i
            lbls = [tgt_v[y, pl.ds(k * _LANES, _LANES)] for k in range(_KX)]
            new = []
            for ci, c in enumerate(grp):
                for k in range(_KX):
                    new.append(jnp.where(lbls[k] == c, 0,
                                         dist[ci * _KX + k] + 1))
            rel = y - row0

            @pl.when(jnp.logical_and(rel >= 0, rel < _RB))
            def _():
                for ci, c in enumerate(grp):
                    for k in range(_KX):
                        sl = pl.ds(((c - 1) * _RB + rel) * _W + k * _LANES,
                                   _LANES)
                        m = jnp.minimum(d1sq_v[sl], new[ci * _KX + k])
                        d1sq_v[sl] = m * m

            return tuple(new)

        lax.fori_loop(0, _H, bwd,
                      tuple(_splat_i32(_FAR) for _ in range(ng * _KX)))

    def smax(yq, carry):
        for k in range(_KX):
            sl = pl.ds(k * _LANES, _LANES)
            ls = [log_v[c, yq, sl] for c in range(_C)]
            m = ls[0]
            for l in ls[1:]:
                m = jnp.maximum(m, l)
            s = jnp.zeros((_LANES,), jnp.float32)
            for l in ls:
                s = s + jnp.exp(l - m)
            maxl_v[yq, sl] = m
            denom_v[yq, sl] = s
            num_v[yq, sl] = jnp.zeros((_LANES,), jnp.float32)
        return carry

    lax.fori_loop(0, _RB, smax, 0)

    for c in range(1, _C):
        def rowgrp(rg, carry):
            yq0 = rg * _RG
            base = ((c - 1) * _RB + yq0) * _W

            def gath(x):
                return [plsc.load_gather(d1sq_v,
                                         [_splat_i32(base + r * _W + x)])
                        for r in range(_RG)]

            def xstep(x, state):
                accs = state[:_RG * _KX]
                bcs = state[_RG * _KX:]
                nxt = gath(x + 1)
                out = []
                for r in range(_RG):
                    for k in range(_KX):
                        out.append(jnp.minimum(
                            accs[r * _KX + k],
                            bcs[r] + dx2_v[x, pl.ds(k * _LANES, _LANES)]))
                return tuple(out) + tuple(nxt)

            state = lax.fori_loop(
                0, _W, xstep,
                tuple(_splat_i32(_ACC0) for _ in range(_RG * _KX))
                + tuple(gath(0)),
                unroll=2)
            accs = state[:_RG * _KX]
            for r in range(_RG):
                yq = yq0 + r
                for k in range(_KX):
                    sl = pl.ds(k * _LANES, _LANES)
                    idx = jnp.minimum(accs[r * _KX + k], _D2MAX + 1)
                    rew = plsc.load_gather(tab_v, [idx])
                    e = jnp.exp(log_v[c, yq, sl] - maxl_v[yq, sl])
                    num_v[yq, sl] = num_v[yq, sl] + e * rew
            return carry

        lax.fori_loop(0, _RB // _RG, rowgrp, 0)

    def fin(yq, accs):
        return tuple(
            accs[k] + num_v[yq, pl.ds(k * _LANES, _LANES)]
            / denom_v[yq, pl.ds(k * _LANES, _LANES)]
            for k in range(_KX))

    accs = lax.fori_loop(0, _RB, fin,
                         tuple(jnp.zeros((_LANES,), jnp.float32)
                               for _ in range(_KX)))
    tot = accs[0]
    for k in range(1, _KX):
        tot = tot + accs[k]
    out_v[...] = tot
    pltpu.sync_copy(out_v, out_hbm.at[wid])


_I = np.arange(_TABN)
_TAB_NP = np.where(_I <= _D2MAX, np.exp(-np.sqrt(_I.astype(np.float32)) / _TAU),
                   0.0).astype(np.float32)
_X = np.arange(_W, dtype=np.int32)
_DX2_NP = ((_X[None, :] - _X[:, None]) ** 2).astype(np.int32)


@jax.jit
def kernel(logits, targets):
    tab = jnp.asarray(_TAB_NP)
    dx2 = jnp.asarray(_DX2_NP)

    mesh = plsc.VectorSubcoreMesh(core_axis_name="c", subcore_axis_name="s")
    run = functools.partial(
        pl.kernel, mesh=mesh,
        compiler_params=pltpu.CompilerParams(needs_layout_passes=False),
        out_type=jax.ShapeDtypeStruct((_NW, _LANES), jnp.float32),
        scratch_types=[
            pltpu.VMEM((_H, _W), jnp.int32),
            pltpu.VMEM((_C, _RB, _W), jnp.float32),
            pltpu.VMEM((_W, _W), jnp.int32),
            pltpu.VMEM((_TABN,), jnp.float32),
            pltpu.VMEM((_NCLS * _RB * _W + 2 * _W,), jnp.int32),
            pltpu.VMEM((_RB, _W), jnp.float32),
            pltpu.VMEM((_RB, _W), jnp.float32),
            pltpu.VMEM((_RB, _W), jnp.float32),
            pltpu.VMEM((_LANES,), jnp.float32),
            pltpu.SemaphoreType.DMA,
        ],
    )(_sc_body)
    partials = run(logits, targets, dx2, tab)
    return -jnp.sum(partials) / (_B * _H * _W)
